# baseline (speedup 1.0000x reference)
import functools

import jax
import jax.numpy as jnp
from jax import lax
from jax.experimental import pallas as pl
from jax.experimental.pallas import tpu as pltpu
from jax.experimental.pallas import tpu_sc as plsc

VOCAB = 1000000
EMBED = 64
BATCH = 4096
SEQ = 200

NC = 2
NS = 16
NW = NC * NS

TOTAL = BATCH * SEQ
PER_W = TOTAL // NW
CHUNK = 128
NCH = PER_W // CHUNK


def _emb_body(table_hbm, idx_hbm, out_hbm, idx_v, rows_v, gsem, psem):
    wid = lax.axis_index("s") * NC + lax.axis_index("c")
    pltpu.sync_copy(idx_hbm.at[wid], idx_v)

    @pl.loop(0, NCH)
    def _(j):
        pltpu.async_copy(table_hbm.at[idx_v.at[j]], rows_v, gsem).wait()
        pltpu.async_copy(rows_v, out_hbm.at[wid, j], psem).wait()


@jax.jit
def _emb(table, idx):
    mesh = plsc.VectorSubcoreMesh(core_axis_name="c", subcore_axis_name="s")
    f = pl.kernel(
        _emb_body,
        out_type=jax.ShapeDtypeStruct((NW, NCH, CHUNK, EMBED), jnp.float32),
        mesh=mesh,
        scratch_types=[
            pltpu.VMEM((NCH, CHUNK), jnp.int32),
            pltpu.VMEM((CHUNK, EMBED), jnp.float32),
            pltpu.SemaphoreType.DMA,
            pltpu.SemaphoreType.DMA,
        ],
        compiler_params=pltpu.CompilerParams(use_tc_tiling_on_sc=False),
    )
    return f(table, idx)


def kernel(input_tokens, table):
    idx = input_tokens.reshape(NW, NCH, CHUNK).astype(jnp.int32)
    out = _emb(table, idx)
    return out.reshape(BATCH, SEQ, EMBED)


# speedup vs baseline: 1.1203x; 1.1203x over previous
import jax
import jax.numpy as jnp
from jax import lax
from jax.experimental import pallas as pl
from jax.experimental.pallas import tpu as pltpu
from jax.experimental.pallas import tpu_sc as plsc

VOCAB = 1000000
EMBED = 64
BATCH = 4096
SEQ = 200

NC = 2
NS = 16
NW = NC * NS

TOTAL = BATCH * SEQ
PER_W = TOTAL // NW
CHUNK = 128
NCH = PER_W // CHUNK

NBUF = 10
DEPTH = 8


def _emb_body(table_hbm, idx_hbm, out_hbm, idx_v, rows_v, gsem, psem):
    wid = lax.axis_index("s") * NC + lax.axis_index("c")
    pltpu.sync_copy(idx_hbm.at[wid], idx_v)

    def fire_gather(j, b):
        pltpu.async_copy(table_hbm.at[idx_v.at[j]], rows_v.at[b], gsem.at[b])

    def wait_gather(b):
        pltpu.make_async_copy(
            table_hbm.at[idx_v.at[0]], rows_v.at[b], gsem.at[b]).wait()

    def fire_put(j, b):
        pltpu.async_copy(rows_v.at[b], out_hbm.at[wid, j], psem.at[b])

    def wait_put(j, b):
        pltpu.make_async_copy(rows_v.at[b], out_hbm.at[wid, j], psem.at[b]).wait()

    for b in range(DEPTH):
        fire_gather(b, b)

    def step(j, b, refill):
        wait_gather(b)
        fire_put(j, b)
        if refill:
            m = j + DEPTH
            bm = (b + DEPTH) % NBUF
            if isinstance(m, int) and m < NBUF:
                pass
            else:
                wait_put(m - NBUF, bm)
            fire_gather(m, bm)

    for j in range(NBUF):
        step(j, j % NBUF, refill=True)

    @pl.loop(NBUF, NCH - NBUF, step=NBUF)
    def _(j0):
        for b in range(NBUF):
            step(j0 + b, b, refill=True)

    for j in range(NCH - NBUF, NCH):
        b = j % NBUF
        wait_gather(b)
        fire_put(j, b)
        m = j + DEPTH
        if m < NCH:
            bm = (b + DEPTH) % NBUF
            wait_put(m - NBUF, bm)
            fire_gather(m, bm)
    for j in range(NCH - NBUF, NCH):
        wait_put(j, j % NBUF)


@jax.jit
def _emb(table, idx):
    mesh = plsc.VectorSubcoreMesh(core_axis_name="c", subcore_axis_name="s")
    f = pl.kernel(
        _emb_body,
        out_type=jax.ShapeDtypeStruct((NW, NCH, CHUNK, EMBED), jnp.float32),
        mesh=mesh,
        scratch_types=[
            pltpu.VMEM((NCH, CHUNK), jnp.int32),
            pltpu.VMEM((NBUF, CHUNK, EMBED), jnp.float32),
            pltpu.SemaphoreType.DMA((NBUF,)),
            pltpu.SemaphoreType.DMA((NBUF,)),
        ],
        compiler_params=pltpu.CompilerParams(use_tc_tiling_on_sc=False),
    )
    return f(table, idx)


def kernel(input_tokens, table):
    idx = input_tokens.reshape(NW, NCH, CHUNK).astype(jnp.int32)
    out = _emb(table, idx)
    return out.reshape(BATCH, SEQ, EMBED)


# speedup vs baseline: 1.1209x; 1.0006x over previous
import jax
import jax.numpy as jnp
from jax import lax
from jax.experimental import pallas as pl
from jax.experimental.pallas import tpu as pltpu
from jax.experimental.pallas import tpu_sc as plsc

VOCAB = 1000000
EMBED = 64
BATCH = 4096
SEQ = 200

NC = 2
NS = 16
NW = NC * NS

ROWS_W = BATCH // NW
NCHUNK = 2 * ROWS_W
SPLIT = (0, 104, 200)

NBUF = 8
DEPTH = 6


def _emb_body(table_hbm, idx_hbm, out_hbm, idx_v, rows_v, gsem, psem):
    wid = lax.axis_index("s") * NC + lax.axis_index("c")
    pltpu.sync_copy(idx_hbm.at[pl.ds(wid * ROWS_W, ROWS_W)], idx_v)

    def parts(j, b):
        h = b % 2
        off, n = SPLIT[h], SPLIT[h + 1] - SPLIT[h]
        return j // 2, off, n

    def fire_gather(j, b, slot):
        r, off, n = parts(j, b)
        pltpu.async_copy(
            table_hbm.at[idx_v.at[r, pl.ds(off, n)]],
            rows_v.at[slot, pl.ds(0, n)],
            gsem.at[slot],
        )

    def wait_gather(j, b, slot):
        _, _, n = parts(j, b)
        pltpu.make_async_copy(
            table_hbm.at[idx_v.at[0, pl.ds(0, n)]],
            rows_v.at[slot, pl.ds(0, n)],
            gsem.at[slot],
        ).wait()

    def fire_put(j, b, slot):
        r, off, n = parts(j, b)
        pltpu.async_copy(
            rows_v.at[slot, pl.ds(0, n)],
            out_hbm.at[wid * ROWS_W + r, pl.ds(off, n)],
            psem.at[slot],
        )

    def wait_put(j, b, slot):
        r, off, n = parts(j, b)
        pltpu.make_async_copy(
            rows_v.at[slot, pl.ds(0, n)],
            out_hbm.at[wid * ROWS_W + r, pl.ds(off, n)],
            psem.at[slot],
        ).wait()

    for c in range(DEPTH):
        fire_gather(c, c, c)

    def step(j, b, refill_static):
        slot = b % NBUF
        wait_gather(j, b, slot)
        fire_put(j, b, slot)
        m = j + DEPTH
        bm = (b + DEPTH) % NBUF
        if refill_static is None or refill_static:
            if isinstance(m, int) and m < NBUF:
                pass
            else:
                wait_put(m - NBUF, bm, bm)
            fire_gather(m, bm, bm)

    for j in range(NBUF):
        step(j, j, True)

    @pl.loop(NBUF, NCHUNK - NBUF, step=NBUF)
    def _(j0):
        for b in range(NBUF):
            step(j0 + b, b, None)

    for j in range(NCHUNK - NBUF, NCHUNK):
        step(j, j % NBUF, j + DEPTH < NCHUNK)

    for j in range(NCHUNK - NBUF, NCHUNK):
        wait_put(j, j % NBUF, j % NBUF)


@jax.jit
def _emb(table, idx):
    mesh = plsc.VectorSubcoreMesh(core_axis_name="c", subcore_axis_name="s")
    f = pl.kernel(
        _emb_body,
        out_type=jax.ShapeDtypeStruct((BATCH, SEQ, EMBED), jnp.float32),
        mesh=mesh,
        scratch_types=[
            pltpu.VMEM((ROWS_W, SEQ), jnp.int32),
            pltpu.VMEM((NBUF, 104, EMBED), jnp.float32),
            pltpu.SemaphoreType.DMA((NBUF,)),
            pltpu.SemaphoreType.DMA((NBUF,)),
        ],
        compiler_params=pltpu.CompilerParams(use_tc_tiling_on_sc=False),
    )
    return f(table, idx)


def kernel(input_tokens, table):
    return _emb(table, input_tokens.astype(jnp.int32))
